# baseline (speedup 1.0000x reference)
import functools

import jax
import jax.numpy as jnp
from jax import lax
from jax.experimental import pallas as pl
from jax.experimental.pallas import tpu as pltpu
from jax.experimental.pallas import tpu_sc as plsc

VOCAB = 1000
VPAD = 1024
NTOK = 1024 * 50


def _lse_body(table_ref, out_ref):
    x = table_ref[...]
    m = jnp.max(x, axis=1, keepdims=True)
    s = jnp.sum(jnp.exp(x - m), axis=1, keepdims=True)
    out_ref[...] = m + jnp.log(s)


def _lse_tc(table):
    v = table.shape[0]
    return pl.pallas_call(
        _lse_body,
        out_shape=jax.ShapeDtypeStruct((v, 1), jnp.float32),
    )(table)


_NC, _NS, _LANES = 2, 16, 16
_NW = _NC * _NS
_BPW = NTOK // _NW
_NBUF = 4
_CHUNK = 16
_NCHUNK = _BPW // _CHUNK


def _sc_gather_build():
    mesh = plsc.VectorSubcoreMesh(core_axis_name="c", subcore_axis_name="s")

    @functools.partial(
        pl.kernel,
        mesh=mesh,
        compiler_params=pltpu.CompilerParams(
            needs_layout_passes=False, use_tc_tiling_on_sc=True
        ),
        out_type=(
            jax.ShapeDtypeStruct((NTOK, VPAD), jnp.float32),
            jax.ShapeDtypeStruct((_NW, _LANES), jnp.float32),
        ),
        scratch_types=[
            pltpu.VMEM((_BPW,), jnp.int32),
            pltpu.VMEM((_BPW,), jnp.int32),
            pltpu.VMEM((_CHUNK, VPAD), jnp.float32),
            pltpu.VMEM((_CHUNK, VPAD), jnp.float32),
            pltpu.VMEM((_CHUNK, VPAD), jnp.float32),
            pltpu.VMEM((_CHUNK, VPAD), jnp.float32),
            pltpu.VMEM((VOCAB,), jnp.float32),
            pltpu.VMEM((_LANES,), jnp.float32),
            pltpu.SemaphoreType.DMA,
            pltpu.SemaphoreType.DMA,
            pltpu.SemaphoreType.DMA,
            pltpu.SemaphoreType.DMA,
            pltpu.SemaphoreType.DMA,
            pltpu.SemaphoreType.DMA,
            pltpu.SemaphoreType.DMA,
            pltpu.SemaphoreType.DMA,
        ],
    )
    def k(idx_hbm, tgt_hbm, lse_hbm, tpad_hbm, out_hbm, part_hbm,
          idx_v, tgt_v, rows0, rows1, rows2, rows3,
          lse_v, acc_v,
          gsem0, gsem1, gsem2, gsem3, wsem0, wsem1, wsem2, wsem3):
        rows = (rows0, rows1, rows2, rows3)
        gsems = (gsem0, gsem1, gsem2, gsem3)
        wsems = (wsem0, wsem1, wsem2, wsem3)
        sid = lax.axis_index("s")
        wid = sid * _NC + lax.axis_index("c")
        base = wid * _BPW
        pltpu.sync_copy(idx_hbm.at[pl.ds(base, _BPW)], idx_v)
        pltpu.sync_copy(tgt_hbm.at[pl.ds(base, _BPW)], tgt_v)
        pltpu.sync_copy(lse_hbm, lse_v)
        acc_v[...] = jnp.zeros((_LANES,), jnp.float32)
        lane = lax.iota(jnp.int32, _LANES)

        def start_gather(g, p):
            pltpu.make_async_copy(
                tpad_hbm.at[idx_v.at[pl.ds(g * _CHUNK, _CHUNK)]],
                rows[p], gsems[p],
            ).start()

        def wait_gather(p):
            pltpu.make_async_copy(
                tpad_hbm.at[idx_v.at[pl.ds(0, _CHUNK)]], rows[p], gsems[p]
            ).wait()

        def start_write(g, p):
            pltpu.make_async_copy(
                rows[p], out_hbm.at[pl.ds(base + g * _CHUNK, _CHUNK)], wsems[p]
            ).start()

        def wait_write(p):
            pltpu.make_async_copy(
                rows[p], out_hbm.at[pl.ds(base, _CHUNK)], wsems[p]
            ).wait()

        def loss_step(g, p):
            sl = pl.ds(g * _CHUNK, _CHUNK)
            c = tgt_v[sl]
            val = plsc.load_gather(rows[p], [lane, c])
            lse_g = plsc.load_gather(lse_v, [idx_v[sl]])
            acc_v[...] = acc_v[...] + (lse_g - val)

        start_gather(0, 0)
        start_gather(1, 1)
        start_gather(2, 2)
        wait_gather(0)
        loss_step(0, 0)
        start_write(0, 0)
        start_gather(3, 3)
        wait_gather(1)
        loss_step(1, 1)
        start_write(1, 1)

        def body(i, carry):
            for q in range(_NBUF):
                g = _NBUF * i + 2 + q
                pw = q
                pg = (2 + q) % _NBUF
                wait_write(pw)
                start_gather(g + 2, pw)
                wait_gather(pg)
                loss_step(g, pg)
                start_write(g, pg)
            return carry

        lax.fori_loop(0, (_NCHUNK - 4) // _NBUF, body, 0)

        wait_gather(2)
        loss_step(_NCHUNK - 2, 2)
        start_write(_NCHUNK - 2, 2)
        wait_gather(3)
        loss_step(_NCHUNK - 1, 3)
        start_write(_NCHUNK - 1, 3)

        for p in range(_NBUF):
            wait_write(p)
        pltpu.sync_copy(acc_v, part_hbm.at[wid])

    return k


_sc_gather = _sc_gather_build()


def _loss_body(part_ref, out_ref):
    out_ref[...] = jnp.sum(part_ref[...], keepdims=True).reshape(1, 1) * (
        1.0 / NTOK
    )


def _loss_tc(partials):
    return pl.pallas_call(
        _loss_body,
        out_shape=jax.ShapeDtypeStruct((1, 1), jnp.float32),
    )(partials)


def kernel(idx, targets, table):
    idx_f = idx.reshape(-1).astype(jnp.int32)
    tgt_f = targets.reshape(-1).astype(jnp.int32)
    lse = _lse_tc(table).reshape(VOCAB)
    tpad = jnp.pad(table, ((0, 0), (0, VPAD - VOCAB)))
    out_pad, partials = _sc_gather(idx_f, tgt_f, lse, tpad)
    loss = _loss_tc(partials)[0, 0]
    return (out_pad[:, :VOCAB], loss)
